# Initial kernel scaffold; baseline (speedup 1.0000x reference)
import functools

import jax
import jax.numpy as jnp
from jax import lax
from jax.experimental import pallas as pl
from jax.experimental.pallas import tpu as pltpu
from jax.experimental.pallas import tpu_sc as plsc

N = 10000
E = 320000
D_IN = 128
DH = 64
DG = 128

NC = 2
NS = 16
NW = NC * NS
EPW = E // NW
C = 80
NCH = EPW // C
G16 = EPW // 16
RPT = N // NS

_MESH = plsc.VectorSubcoreMesh(
    core_axis_name="c", subcore_axis_name="s", num_cores=NC, num_subcores=NS
)


def _wid():
    return lax.axis_index("s") * NC + lax.axis_index("c")


@functools.partial(
    pl.kernel,
    out_type=jax.ShapeDtypeStruct((NW, N), jnp.float32),
    scratch_types=[
        pltpu.VMEM((G16, 16), jnp.int32),
        pltpu.VMEM((G16, 16), jnp.float32),
        pltpu.VMEM((N,), jnp.float32),
    ],
    mesh=_MESH,
)
def _sc_deg(cidx_hbm, ew_hbm, zn_hbm, degp_hbm, cidx_v, ew_v, acc_v):
    wid = _wid()
    pltpu.sync_copy(cidx_hbm.at[wid], cidx_v)
    pltpu.sync_copy(ew_hbm.at[wid], ew_v)
    pltpu.sync_copy(zn_hbm, acc_v)

    def body(g, carry):
        c16 = cidx_v[g]
        w16 = ew_v[g]
        plsc.addupdate_scatter(acc_v, [c16], w16)
        return carry

    lax.fori_loop(0, G16, body, 0)
    pltpu.sync_copy(acc_v, degp_hbm.at[wid])


@functools.partial(
    pl.kernel,
    out_type=jax.ShapeDtypeStruct((NW, G16, 16), jnp.float32),
    scratch_types=[
        pltpu.VMEM((G16, 16), jnp.int32),
        pltpu.VMEM((G16, 16), jnp.int32),
        pltpu.VMEM((G16, 16), jnp.float32),
        pltpu.VMEM((G16, 16), jnp.float32),
        pltpu.VMEM((N,), jnp.float32),
    ],
    mesh=_MESH,
)
def _sc_norm(ridx_hbm, cidx_hbm, ew_hbm, dinv_hbm, norm_hbm,
             ridx_v, cidx_v, ew_v, norm_v, dinv_v):
    wid = _wid()
    pltpu.sync_copy(ridx_hbm.at[wid], ridx_v)
    pltpu.sync_copy(cidx_hbm.at[wid], cidx_v)
    pltpu.sync_copy(ew_hbm.at[wid], ew_v)
    pltpu.sync_copy(dinv_hbm, dinv_v)

    def body(g, carry):
        r16 = ridx_v[g]
        c16 = cidx_v[g]
        nrm = plsc.load_gather(dinv_v, [r16]) * ew_v[g] * plsc.load_gather(dinv_v, [c16])
        norm_v[g] = nrm
        return carry

    lax.fori_loop(0, G16, body, 0)
    pltpu.sync_copy(norm_v, norm_hbm.at[wid])


@functools.partial(
    pl.kernel,
    out_type=[
        jax.ShapeDtypeStruct((NW, G16, 16), jnp.float32),
        jax.ShapeDtypeStruct((NW, N), jnp.float32),
    ],
    scratch_types=[
        pltpu.VMEM((G16, 16), jnp.int32),
        pltpu.VMEM((G16, 16), jnp.int32),
        pltpu.VMEM((G16, 16), jnp.float32),
        pltpu.VMEM((N,), jnp.float32),
        pltpu.VMEM((N,), jnp.float32),
        pltpu.VMEM((N,), jnp.float32),
        pltpu.VMEM((N,), jnp.float32),
    ],
    mesh=_MESH,
)
def _sc_att(ridx_hbm, cidx_hbm, asrc_hbm, adst_hbm, p_hbm, zn_hbm,
            e_hbm, sp_hbm,
            ridx_v, cidx_v, e_v, as_v, ad_v, p_v, s_v):
    wid = _wid()
    pltpu.sync_copy(ridx_hbm.at[wid], ridx_v)
    pltpu.sync_copy(cidx_hbm.at[wid], cidx_v)
    pltpu.sync_copy(asrc_hbm, as_v)
    pltpu.sync_copy(adst_hbm, ad_v)
    pltpu.sync_copy(p_hbm, p_v)
    pltpu.sync_copy(zn_hbm, s_v)

    def body(g, carry):
        r16 = ridx_v[g]
        c16 = cidx_v[g]
        al = plsc.load_gather(as_v, [r16]) + plsc.load_gather(ad_v, [c16])
        al = jnp.where(al > 0, al, 0.2 * al)
        ee = jnp.exp(al - plsc.load_gather(p_v, [c16]))
        e_v[g] = ee
        plsc.addupdate_scatter(s_v, [c16], ee)
        return carry

    lax.fori_loop(0, G16, body, 0)
    pltpu.sync_copy(e_v, e_hbm.at[wid])
    pltpu.sync_copy(s_v, sp_hbm.at[wid])


def _make_row_pass(D, mul_pernode):
    @functools.partial(
        pl.kernel,
        out_type=jax.ShapeDtypeStruct((NC, N, D), jnp.float32),
        scratch_types=[
            pltpu.VMEM((NCH, C), jnp.int32),
            pltpu.VMEM((NCH, C), jnp.int32),
            pltpu.VMEM((NCH, C), jnp.float32),
            pltpu.VMEM((N,), jnp.float32),
            pltpu.VMEM((C, D), jnp.float32),
            pltpu.VMEM_SHARED((N, D), jnp.float32),
        ],
        mesh=_MESH,
    )
    def row_pass(h_hbm, ridx_hbm, cidx_hbm, coef_hbm, pn_hbm, zd_hbm,
                 accp_hbm, ridx_v, cidx_v, coef_v, pn_v, buf, acc_sh):
        cid = lax.axis_index("c")
        sid = lax.axis_index("s")
        wid = sid * NC + cid
        pltpu.sync_copy(zd_hbm.at[pl.ds(sid * RPT, RPT)],
                        acc_sh.at[pl.ds(sid * RPT, RPT)])
        pltpu.sync_copy(ridx_hbm.at[wid], ridx_v)
        pltpu.sync_copy(cidx_hbm.at[wid], cidx_v)
        pltpu.sync_copy(coef_hbm.at[wid], coef_v)
        pltpu.sync_copy(pn_hbm, pn_v)
        plsc.subcore_barrier()

        def chunk(j, carry):
            if mul_pernode:
                for g in range(C // 16):
                    c16 = cidx_v[j, pl.ds(g * 16, 16)]
                    coef_v[j, pl.ds(g * 16, 16)] = (
                        coef_v[j, pl.ds(g * 16, 16)]
                        * plsc.load_gather(pn_v, [c16])
                    )
            pltpu.sync_copy(h_hbm.at[ridx_v.at[j]], buf)
            for g in range(C // 16):
                nv = coef_v[j, pl.ds(g * 16, 16)]
                for e16 in range(16):
                    b = jnp.take(nv, jnp.full((16,), e16, jnp.int32),
                                 mode="promise_in_bounds")
                    e = g * 16 + e16
                    for k in range(D // 16):
                        buf[e, pl.ds(k * 16, 16)] = buf[e, pl.ds(k * 16, 16)] * b
            pltpu.sync_copy(buf, acc_sh.at[cidx_v.at[j]], add=True)
            return carry

        lax.fori_loop(0, NCH, chunk, 0)
        plsc.subcore_barrier()
        pltpu.sync_copy(acc_sh.at[pl.ds(sid * RPT, RPT)],
                        accp_hbm.at[cid, pl.ds(sid * RPT, RPT)])

    return row_pass


_row_gcn = _make_row_pass(DH, False)
_row_gat = _make_row_pass(DG, True)


def _tc(body, out_shape, *args):
    return pl.pallas_call(body, out_shape=out_shape)(*args)


def _mm_body(x_ref, w_ref, o_ref):
    o_ref[...] = jnp.dot(x_ref[...], w_ref[...],
                         preferred_element_type=jnp.float32)


def _dinv_body(degp_ref, o_ref):
    deg = jnp.sum(degp_ref[...], axis=0) + 1.0
    o_ref[...] = lax.rsqrt(deg)


def _comb_mm_body(accp_ref, h_ref, dinv_ref, b_ref, w_ref, o_ref):
    dinv2 = (dinv_ref[...] * dinv_ref[...])[:, None]
    out = accp_ref[0] + accp_ref[1] + dinv2 * h_ref[...] + b_ref[...][None, :]
    o_ref[...] = jnp.dot(out, w_ref[...], preferred_element_type=jnp.float32)


def _gat_mm_body(accp_ref, h_ref, dinv_ref, b_ref, w_ref, asrc_ref, adst_ref,
                 hg_ref, as_ref, ad_ref):
    dinv2 = (dinv_ref[...] * dinv_ref[...])[:, None]
    out = accp_ref[0] + accp_ref[1] + dinv2 * h_ref[...] + b_ref[...][None, :]
    hg = jnp.dot(out, w_ref[...], preferred_element_type=jnp.float32)
    hg_ref[...] = hg
    as_ref[...] = jnp.dot(hg, asrc_ref[...], preferred_element_type=jnp.float32)
    ad_ref[...] = jnp.dot(hg, adst_ref[...], preferred_element_type=jnp.float32)


def _shift_body(as_ref, ad_ref, p_ref, eself_ref):
    amax = jnp.max(as_ref[...])
    pa = amax + ad_ref[...]
    p = jnp.where(pa > 0, pa, 0.2 * pa)
    p_ref[...] = p
    al = as_ref[...] + ad_ref[...]
    al = jnp.where(al > 0, al, 0.2 * al)
    eself_ref[...] = jnp.exp(al - p)


def _sinv_body(sp_ref, eself_ref, sinv_ref, cself_ref):
    s = jnp.sum(sp_ref[...], axis=0) + eself_ref[...]
    sinv = 1.0 / (s + 1e-16)
    sinv_ref[...] = sinv
    cself_ref[...] = eself_ref[...] * sinv


def _final_body(accp_ref, hg_ref, cself_ref, bg_ref, o_ref):
    o_ref[...] = (accp_ref[0] + accp_ref[1]
                  + cself_ref[...][:, None] * hg_ref[...]
                  + bg_ref[...][None, :])


def kernel(x, edge_index, edge_attr, W1, b1, W2, b2, W3, b3, Wg,
           att_src, att_dst, bg):
    f32 = jnp.float32
    row = edge_index[0]
    col = edge_index[1]
    ridx_s = row.reshape(NW, G16, 16)
    cidx_s = col.reshape(NW, G16, 16)
    ew_s = edge_attr.reshape(NW, G16, 16)
    ridx_r = row.reshape(NW, NCH, C)
    cidx_r = col.reshape(NW, NCH, C)
    zn = jnp.zeros((N,), f32)
    z64 = jnp.zeros((N, DH), f32)
    z128 = jnp.zeros((N, DG), f32)

    degp = _sc_deg(cidx_s, ew_s, zn)
    dinv = _tc(_dinv_body, jax.ShapeDtypeStruct((N,), f32), degp)

    norm_s = _sc_norm(ridx_s, cidx_s, ew_s, dinv)
    norm_r = norm_s.reshape(NW, NCH, C)

    h1 = _tc(_mm_body, jax.ShapeDtypeStruct((N, DH), f32), x, W1)
    acc1 = _row_gcn(h1, ridx_r, cidx_r, norm_r, zn, z64)
    h2 = _tc(_comb_mm_body, jax.ShapeDtypeStruct((N, DH), f32),
             acc1, h1, dinv, b1, W2)
    acc2 = _row_gcn(h2, ridx_r, cidx_r, norm_r, zn, z64)
    h3 = _tc(_comb_mm_body, jax.ShapeDtypeStruct((N, DH), f32),
             acc2, h2, dinv, b2, W3)
    acc3 = _row_gcn(h3, ridx_r, cidx_r, norm_r, zn, z64)

    hg, a_s, a_d = _tc(_gat_mm_body,
                       [jax.ShapeDtypeStruct((N, DG), f32),
                        jax.ShapeDtypeStruct((N,), f32),
                        jax.ShapeDtypeStruct((N,), f32)],
                       acc3, h3, dinv, b3, Wg, att_src, att_dst)
    p, e_self = _tc(_shift_body,
                    [jax.ShapeDtypeStruct((N,), f32),
                     jax.ShapeDtypeStruct((N,), f32)],
                    a_s, a_d)
    e_s, s_p = _sc_att(ridx_s, cidx_s, a_s, a_d, p, zn)
    sinv, cself = _tc(_sinv_body,
                      [jax.ShapeDtypeStruct((N,), f32),
                       jax.ShapeDtypeStruct((N,), f32)],
                      s_p, e_self)
    e_r = e_s.reshape(NW, NCH, C)
    accg = _row_gat(hg, ridx_r, cidx_r, e_r, sinv, z128)
    out = _tc(_final_body, jax.ShapeDtypeStruct((N, DG), f32),
              accg, hg, cself, bg)
    return out


# speedup vs baseline: 9.8351x; 9.8351x over previous
import functools

import jax
import jax.numpy as jnp
from jax import lax
from jax.experimental import pallas as pl
from jax.experimental.pallas import tpu as pltpu
from jax.experimental.pallas import tpu_sc as plsc

N = 10000
E = 320000
D_IN = 128
DH = 64
DG = 128

NC = 2
NS = 16
NW = NC * NS
EPW = E // NW
G16 = EPW // 16
RPT = 624
SR = 104

_MESH = plsc.VectorSubcoreMesh(
    core_axis_name="c", subcore_axis_name="s", num_cores=NC, num_subcores=NS
)
_CP = pltpu.CompilerParams(needs_layout_passes=False)
_f32 = jnp.float32


def _g16(ref, g):
    return ref[pl.ds(pl.multiple_of(g * 16, 16), 16)]


def _eslice(ref, wid):
    return ref.at[pl.ds(wid * EPW, EPW)]


@functools.partial(
    pl.kernel,
    out_type=jax.ShapeDtypeStruct((NW * N,), _f32),
    scratch_types=[
        pltpu.VMEM((EPW,), jnp.int32),
        pltpu.VMEM((EPW,), _f32),
        pltpu.VMEM((N,), _f32),
    ],
    mesh=_MESH,
    compiler_params=_CP,
)
def _sc_deg(cidx_hbm, ew_hbm, zn_hbm, degp_hbm, cidx_v, ew_v, acc_v):
    wid = lax.axis_index("s") * NC + lax.axis_index("c")
    pltpu.sync_copy(_eslice(cidx_hbm, wid), cidx_v)
    pltpu.sync_copy(_eslice(ew_hbm, wid), ew_v)
    pltpu.sync_copy(zn_hbm, acc_v)

    def body(g, carry):
        plsc.addupdate_scatter(acc_v, [_g16(cidx_v, g)], _g16(ew_v, g))
        return carry

    lax.fori_loop(0, G16, body, 0)
    pltpu.sync_copy(acc_v, degp_hbm.at[pl.ds(wid * N, N)])


@functools.partial(
    pl.kernel,
    out_type=jax.ShapeDtypeStruct((E,), _f32),
    scratch_types=[
        pltpu.VMEM((EPW,), jnp.int32),
        pltpu.VMEM((EPW,), jnp.int32),
        pltpu.VMEM((EPW,), _f32),
        pltpu.VMEM((EPW,), _f32),
        pltpu.VMEM((N,), _f32),
    ],
    mesh=_MESH,
    compiler_params=_CP,
)
def _sc_norm(ridx_hbm, cidx_hbm, ew_hbm, dinv_hbm, norm_hbm,
             ridx_v, cidx_v, ew_v, norm_v, dinv_v):
    wid = lax.axis_index("s") * NC + lax.axis_index("c")
    pltpu.sync_copy(_eslice(ridx_hbm, wid), ridx_v)
    pltpu.sync_copy(_eslice(cidx_hbm, wid), cidx_v)
    pltpu.sync_copy(_eslice(ew_hbm, wid), ew_v)
    pltpu.sync_copy(dinv_hbm, dinv_v)

    def body(g, carry):
        nrm = (plsc.load_gather(dinv_v, [_g16(ridx_v, g)])
               * _g16(ew_v, g)
               * plsc.load_gather(dinv_v, [_g16(cidx_v, g)]))
        norm_v[pl.ds(pl.multiple_of(g * 16, 16), 16)] = nrm
        return carry

    lax.fori_loop(0, G16, body, 0)
    pltpu.sync_copy(norm_v, _eslice(norm_hbm, wid))


@functools.partial(
    pl.kernel,
    out_type=[
        jax.ShapeDtypeStruct((E,), _f32),
        jax.ShapeDtypeStruct((NW * N,), _f32),
    ],
    scratch_types=[
        pltpu.VMEM((EPW,), jnp.int32),
        pltpu.VMEM((EPW,), jnp.int32),
        pltpu.VMEM((EPW,), _f32),
        pltpu.VMEM((N,), _f32),
        pltpu.VMEM((N,), _f32),
        pltpu.VMEM((N,), _f32),
        pltpu.VMEM((N,), _f32),
    ],
    mesh=_MESH,
    compiler_params=_CP,
)
def _sc_att(ridx_hbm, cidx_hbm, asrc_hbm, adst_hbm, p_hbm, zn_hbm,
            e_hbm, sp_hbm,
            ridx_v, cidx_v, e_v, as_v, ad_v, p_v, s_v):
    wid = lax.axis_index("s") * NC + lax.axis_index("c")
    pltpu.sync_copy(_eslice(ridx_hbm, wid), ridx_v)
    pltpu.sync_copy(_eslice(cidx_hbm, wid), cidx_v)
    pltpu.sync_copy(asrc_hbm, as_v)
    pltpu.sync_copy(adst_hbm, ad_v)
    pltpu.sync_copy(p_hbm, p_v)
    pltpu.sync_copy(zn_hbm, s_v)

    def body(g, carry):
        r16 = _g16(ridx_v, g)
        c16 = _g16(cidx_v, g)
        al = plsc.load_gather(as_v, [r16]) + plsc.load_gather(ad_v, [c16])
        al = jnp.where(al > 0, al, 0.2 * al)
        ee = jnp.exp(al - plsc.load_gather(p_v, [c16]))
        e_v[pl.ds(pl.multiple_of(g * 16, 16), 16)] = ee
        plsc.addupdate_scatter(s_v, [c16], ee)
        return carry

    lax.fori_loop(0, G16, body, 0)
    pltpu.sync_copy(e_v, _eslice(e_hbm, wid))
    pltpu.sync_copy(s_v, sp_hbm.at[pl.ds(wid * N, N)])


@functools.partial(
    pl.kernel,
    out_type=jax.ShapeDtypeStruct((E,), _f32),
    scratch_types=[
        pltpu.VMEM((EPW,), jnp.int32),
        pltpu.VMEM((EPW,), _f32),
        pltpu.VMEM((EPW,), _f32),
        pltpu.VMEM((N,), _f32),
    ],
    mesh=_MESH,
    compiler_params=_CP,
)
def _sc_coef(cidx_hbm, e_hbm, sinv_hbm, coef_hbm, cidx_v, e_v, coef_v, sinv_v):
    wid = lax.axis_index("s") * NC + lax.axis_index("c")
    pltpu.sync_copy(_eslice(cidx_hbm, wid), cidx_v)
    pltpu.sync_copy(_eslice(e_hbm, wid), e_v)
    pltpu.sync_copy(sinv_hbm, sinv_v)

    def body(g, carry):
        cf = _g16(e_v, g) * plsc.load_gather(sinv_v, [_g16(cidx_v, g)])
        coef_v[pl.ds(pl.multiple_of(g * 16, 16), 16)] = cf
        return carry

    lax.fori_loop(0, G16, body, 0)
    pltpu.sync_copy(coef_v, _eslice(coef_hbm, wid))


@functools.partial(
    pl.kernel,
    out_type=jax.ShapeDtypeStruct((NC * N, DG), _f32),
    scratch_types=[
        pltpu.VMEM((EPW,), jnp.int32),
        pltpu.VMEM((EPW,), jnp.int32),
        pltpu.VMEM((EPW,), _f32),
        pltpu.VMEM((16, DG), _f32),
        pltpu.VMEM((SR, DG), _f32),
        pltpu.VMEM_SHARED((N, DG), _f32),
    ],
    mesh=_MESH,
    compiler_params=_CP,
)
def _sc_row(h_hbm, ridx_hbm, cidx_hbm, coef_hbm, accp_hbm,
            ridx_v, cidx_v, coef_v, buf, stage, acc_sh):
    cid = lax.axis_index("c")
    sid = lax.axis_index("s")
    wid = sid * NC + cid

    def zrow(i, carry):
        for k in range(DG // 16):
            stage[i, pl.ds(k * 16, 16)] = jnp.zeros((16,), _f32)
        return carry

    lax.fori_loop(0, SR, zrow, 0)
    for t in range(6):
        r0 = pl.multiple_of(sid * RPT + t * SR, 8)
        pltpu.sync_copy(stage, acc_sh.at[pl.ds(r0, SR)])

    @pl.when(sid == 0)
    def _():
        pltpu.sync_copy(stage.at[pl.ds(0, 16)],
                        acc_sh.at[pl.ds(NS * RPT, N - NS * RPT)])

    pltpu.sync_copy(_eslice(ridx_hbm, wid), ridx_v)
    pltpu.sync_copy(_eslice(cidx_hbm, wid), cidx_v)
    pltpu.sync_copy(_eslice(coef_hbm, wid), coef_v)
    plsc.subcore_barrier()

    def body(g, carry):
        r16 = _g16(ridx_v, g)
        c16 = _g16(cidx_v, g)
        nv = _g16(coef_v, g)
        pltpu.sync_copy(h_hbm.at[r16], buf)
        for e16 in range(16):
            b = nv.at[jnp.full((16,), e16, jnp.int32)].get(
                mode="promise_in_bounds")
            for k in range(DG // 16):
                buf[e16, pl.ds(k * 16, 16)] = buf[e16, pl.ds(k * 16, 16)] * b
        pltpu.sync_copy(buf, acc_sh.at[c16], add=True)
        return carry

    lax.fori_loop(0, G16, body, 0)
    plsc.subcore_barrier()
    for t in range(6):
        r0 = pl.multiple_of(sid * RPT + t * SR, 8)
        pltpu.sync_copy(acc_sh.at[pl.ds(r0, SR)], stage)
        pltpu.sync_copy(stage, accp_hbm.at[pl.ds(cid * N + r0, SR)])

    @pl.when(sid == 0)
    def _():
        tail = pl.multiple_of(NS * RPT, 8)
        pltpu.sync_copy(acc_sh.at[pl.ds(tail, N - NS * RPT)],
                        stage.at[pl.ds(0, 16)])
        pltpu.sync_copy(stage.at[pl.ds(0, 16)],
                        accp_hbm.at[pl.ds(cid * N + tail, N - NS * RPT)])


def _tc(body, out_shape, *args):
    return pl.pallas_call(body, out_shape=out_shape)(*args)


def _mm_body(x_ref, w_ref, o_ref):
    o_ref[...] = jnp.dot(x_ref[...], w_ref[...], preferred_element_type=_f32)


def _dinv_body(degp_ref, o_ref):
    deg = jnp.sum(degp_ref[...], axis=0) + 1.0
    o_ref[...] = lax.rsqrt(deg)


def _comb_mm_body(accp_ref, h_ref, dinv_ref, b_ref, w_ref, o_ref):
    dinv2 = (dinv_ref[...] * dinv_ref[...])[:, None]
    out = accp_ref[0] + accp_ref[1] + dinv2 * h_ref[...] + b_ref[...][None, :]
    o_ref[...] = jnp.dot(out, w_ref[...], preferred_element_type=_f32)


def _gat_mm_body(accp_ref, h_ref, dinv_ref, b_ref, w_ref, asrc_ref, adst_ref,
                 hg_ref, as_ref, ad_ref):
    dinv2 = (dinv_ref[...] * dinv_ref[...])[:, None]
    out = accp_ref[0] + accp_ref[1] + dinv2 * h_ref[...] + b_ref[...][None, :]
    hg = jnp.dot(out, w_ref[...], preferred_element_type=_f32)
    hg_ref[...] = hg
    as_ref[...] = jnp.dot(hg, asrc_ref[...], preferred_element_type=_f32)
    ad_ref[...] = jnp.dot(hg, adst_ref[...], preferred_element_type=_f32)


def _shift_body(as_ref, ad_ref, p_ref, eself_ref):
    amax = jnp.max(as_ref[...])
    pa = amax + ad_ref[...]
    p = jnp.where(pa > 0, pa, 0.2 * pa)
    p_ref[...] = p
    al = as_ref[...] + ad_ref[...]
    al = jnp.where(al > 0, al, 0.2 * al)
    eself_ref[...] = jnp.exp(al - p)


def _sinv_body(sp_ref, eself_ref, sinv_ref, cself_ref):
    s = jnp.sum(sp_ref[...], axis=0) + eself_ref[...]
    sinv = 1.0 / (s + 1e-16)
    sinv_ref[...] = sinv
    cself_ref[...] = eself_ref[...] * sinv


def _final_body(accg_ref, hg_ref, cself_ref, bg_ref, o_ref):
    o_ref[...] = (accg_ref[0] + accg_ref[1]
                  + cself_ref[...][:, None] * hg_ref[...]
                  + bg_ref[...][None, :])


def kernel(x, edge_index, edge_attr, W1, b1, W2, b2, W3, b3, Wg,
           att_src, att_dst, bg):
    ridx = edge_index[0]
    cidx = edge_index[1]
    zn = jnp.zeros((N,), _f32)
    pad = DG - DH
    W1p = jnp.pad(W1, ((0, 0), (0, pad)))
    W2p = jnp.pad(W2, ((0, pad), (0, pad)))
    W3p = jnp.pad(W3, ((0, pad), (0, pad)))
    Wgp = jnp.pad(Wg, ((0, pad), (0, 0)))
    b1p = jnp.pad(b1, (0, pad))
    b2p = jnp.pad(b2, (0, pad))
    b3p = jnp.pad(b3, (0, pad))

    degp = _sc_deg(cidx, edge_attr, zn).reshape(NW, N)
    dinv = _tc(_dinv_body, jax.ShapeDtypeStruct((N,), _f32), degp)

    norm = _sc_norm(ridx, cidx, edge_attr, dinv)

    h1 = _tc(_mm_body, jax.ShapeDtypeStruct((N, DG), _f32), x, W1p)
    acc1 = _sc_row(h1, ridx, cidx, norm).reshape(NC, N, DG)
    h2 = _tc(_comb_mm_body, jax.ShapeDtypeStruct((N, DG), _f32),
             acc1, h1, dinv, b1p, W2p)
    acc2 = _sc_row(h2, ridx, cidx, norm).reshape(NC, N, DG)
    h3 = _tc(_comb_mm_body, jax.ShapeDtypeStruct((N, DG), _f32),
             acc2, h2, dinv, b2p, W3p)
    acc3 = _sc_row(h3, ridx, cidx, norm).reshape(NC, N, DG)

    hg, a_s, a_d = _tc(_gat_mm_body,
                       [jax.ShapeDtypeStruct((N, DG), _f32),
                        jax.ShapeDtypeStruct((N,), _f32),
                        jax.ShapeDtypeStruct((N,), _f32)],
                       acc3, h3, dinv, b3p, Wgp, att_src, att_dst)
    p, e_self = _tc(_shift_body,
                    [jax.ShapeDtypeStruct((N,), _f32),
                     jax.ShapeDtypeStruct((N,), _f32)],
                    a_s, a_d)
    e_s, s_p = _sc_att(ridx, cidx, a_s, a_d, p, zn)
    sinv, cself = _tc(_sinv_body,
                      [jax.ShapeDtypeStruct((N,), _f32),
                       jax.ShapeDtypeStruct((N,), _f32)],
                      s_p.reshape(NW, N), e_self)
    coefg = _sc_coef(cidx, e_s, sinv)
    accg = _sc_row(hg, ridx, cidx, coefg).reshape(NC, N, DG)
    out = _tc(_final_body, jax.ShapeDtypeStruct((N, DG), _f32),
              accg, hg, cself, bg)
    return out


# speedup vs baseline: 17.0043x; 1.7289x over previous
import functools

import jax
import jax.numpy as jnp
from jax import lax
from jax.experimental import pallas as pl
from jax.experimental.pallas import tpu as pltpu
from jax.experimental.pallas import tpu_sc as plsc

N = 10000
E = 320000
D_IN = 128
DH = 64
DG = 128

NC = 2
NS = 16
NW = NC * NS
EPW = E // NW
G16 = EPW // 16
CH = 80
NCHK = EPW // CH
RPT = 624
SR = 104

_MESH = plsc.VectorSubcoreMesh(
    core_axis_name="c", subcore_axis_name="s", num_cores=NC, num_subcores=NS
)
_CP = pltpu.CompilerParams(needs_layout_passes=False)
_f32 = jnp.float32


def _g16(ref, g):
    return ref[pl.ds(pl.multiple_of(g * 16, 16), 16)]


def _eslice(ref, wid):
    return ref.at[pl.ds(wid * EPW, EPW)]


@functools.partial(
    pl.kernel,
    out_type=jax.ShapeDtypeStruct((NW * N,), _f32),
    scratch_types=[
        pltpu.VMEM((EPW,), jnp.int32),
        pltpu.VMEM((EPW,), _f32),
        pltpu.VMEM((N,), _f32),
    ],
    mesh=_MESH,
    compiler_params=_CP,
)
def _sc_deg(cidx_hbm, ew_hbm, zn_hbm, degp_hbm, cidx_v, ew_v, acc_v):
    wid = lax.axis_index("s") * NC + lax.axis_index("c")
    pltpu.sync_copy(_eslice(cidx_hbm, wid), cidx_v)
    pltpu.sync_copy(_eslice(ew_hbm, wid), ew_v)
    pltpu.sync_copy(zn_hbm, acc_v)

    def body(g, carry):
        plsc.addupdate_scatter(acc_v, [_g16(cidx_v, g)], _g16(ew_v, g))
        return carry

    lax.fori_loop(0, G16, body, 0)
    pltpu.sync_copy(acc_v, degp_hbm.at[pl.ds(wid * N, N)])


@functools.partial(
    pl.kernel,
    out_type=jax.ShapeDtypeStruct((E,), _f32),
    scratch_types=[
        pltpu.VMEM((EPW,), jnp.int32),
        pltpu.VMEM((EPW,), jnp.int32),
        pltpu.VMEM((EPW,), _f32),
        pltpu.VMEM((EPW,), _f32),
        pltpu.VMEM((N,), _f32),
    ],
    mesh=_MESH,
    compiler_params=_CP,
)
def _sc_norm(ridx_hbm, cidx_hbm, ew_hbm, dinv_hbm, norm_hbm,
             ridx_v, cidx_v, ew_v, norm_v, dinv_v):
    wid = lax.axis_index("s") * NC + lax.axis_index("c")
    pltpu.sync_copy(_eslice(ridx_hbm, wid), ridx_v)
    pltpu.sync_copy(_eslice(cidx_hbm, wid), cidx_v)
    pltpu.sync_copy(_eslice(ew_hbm, wid), ew_v)
    pltpu.sync_copy(dinv_hbm, dinv_v)

    def body(g, carry):
        nrm = (plsc.load_gather(dinv_v, [_g16(ridx_v, g)])
               * _g16(ew_v, g)
               * plsc.load_gather(dinv_v, [_g16(cidx_v, g)]))
        norm_v[pl.ds(pl.multiple_of(g * 16, 16), 16)] = nrm
        return carry

    lax.fori_loop(0, G16, body, 0)
    pltpu.sync_copy(norm_v, _eslice(norm_hbm, wid))


@functools.partial(
    pl.kernel,
    out_type=[
        jax.ShapeDtypeStruct((E,), _f32),
        jax.ShapeDtypeStruct((NW * N,), _f32),
    ],
    scratch_types=[
        pltpu.VMEM((EPW,), jnp.int32),
        pltpu.VMEM((EPW,), jnp.int32),
        pltpu.VMEM((EPW,), _f32),
        pltpu.VMEM((N,), _f32),
        pltpu.VMEM((N,), _f32),
        pltpu.VMEM((N,), _f32),
        pltpu.VMEM((N,), _f32),
    ],
    mesh=_MESH,
    compiler_params=_CP,
)
def _sc_att(ridx_hbm, cidx_hbm, asrc_hbm, adst_hbm, p_hbm, zn_hbm,
            e_hbm, sp_hbm,
            ridx_v, cidx_v, e_v, as_v, ad_v, p_v, s_v):
    wid = lax.axis_index("s") * NC + lax.axis_index("c")
    pltpu.sync_copy(_eslice(ridx_hbm, wid), ridx_v)
    pltpu.sync_copy(_eslice(cidx_hbm, wid), cidx_v)
    pltpu.sync_copy(asrc_hbm, as_v)
    pltpu.sync_copy(adst_hbm, ad_v)
    pltpu.sync_copy(p_hbm, p_v)
    pltpu.sync_copy(zn_hbm, s_v)

    def body(g, carry):
        r16 = _g16(ridx_v, g)
        c16 = _g16(cidx_v, g)
        al = plsc.load_gather(as_v, [r16]) + plsc.load_gather(ad_v, [c16])
        al = jnp.where(al > 0, al, 0.2 * al)
        ee = jnp.exp(al - plsc.load_gather(p_v, [c16]))
        e_v[pl.ds(pl.multiple_of(g * 16, 16), 16)] = ee
        plsc.addupdate_scatter(s_v, [c16], ee)
        return carry

    lax.fori_loop(0, G16, body, 0)
    pltpu.sync_copy(e_v, _eslice(e_hbm, wid))
    pltpu.sync_copy(s_v, sp_hbm.at[pl.ds(wid * N, N)])


@functools.partial(
    pl.kernel,
    out_type=jax.ShapeDtypeStruct((E,), _f32),
    scratch_types=[
        pltpu.VMEM((EPW,), jnp.int32),
        pltpu.VMEM((EPW,), _f32),
        pltpu.VMEM((EPW,), _f32),
        pltpu.VMEM((N,), _f32),
    ],
    mesh=_MESH,
    compiler_params=_CP,
)
def _sc_coef(cidx_hbm, e_hbm, sinv_hbm, coef_hbm, cidx_v, e_v, coef_v, sinv_v):
    wid = lax.axis_index("s") * NC + lax.axis_index("c")
    pltpu.sync_copy(_eslice(cidx_hbm, wid), cidx_v)
    pltpu.sync_copy(_eslice(e_hbm, wid), e_v)
    pltpu.sync_copy(sinv_hbm, sinv_v)

    def body(g, carry):
        cf = _g16(e_v, g) * plsc.load_gather(sinv_v, [_g16(cidx_v, g)])
        coef_v[pl.ds(pl.multiple_of(g * 16, 16), 16)] = cf
        return carry

    lax.fori_loop(0, G16, body, 0)
    pltpu.sync_copy(coef_v, _eslice(coef_hbm, wid))


@functools.partial(
    pl.kernel,
    out_type=jax.ShapeDtypeStruct((NC * N, DG), _f32),
    scratch_types=[
        pltpu.VMEM((CH,), jnp.int32),
        pltpu.VMEM((CH,), jnp.int32),
        pltpu.VMEM((CH,), jnp.int32),
        pltpu.VMEM((CH,), jnp.int32),
        pltpu.VMEM((CH,), _f32),
        pltpu.VMEM((CH,), _f32),
        pltpu.VMEM((CH, DG), _f32),
        pltpu.VMEM((CH, DG), _f32),
        pltpu.VMEM((SR, DG), _f32),
        pltpu.SemaphoreType.DMA,
        pltpu.SemaphoreType.DMA,
        pltpu.VMEM_SHARED((N, DG), _f32),
    ],
    mesh=_MESH,
    compiler_params=_CP,
)
def _sc_row(h_hbm, ridx_hbm, cidx_hbm, coef_hbm, accp_hbm,
            ridx0, ridx1, cidx0, cidx1, coef0, coef1, rows0, rows1,
            stage, sem0, sem1, acc_sh):
    cid = lax.axis_index("c")
    sid = lax.axis_index("s")
    wid = sid * NC + cid
    ebase = wid * EPW
    rbuf = (ridx0, ridx1)
    cbuf = (cidx0, cidx1)
    fbuf = (coef0, coef1)
    rows = (rows0, rows1)
    sems = (sem0, sem1)

    def zrow(i, carry):
        for k in range(DG // 16):
            stage[i, pl.ds(k * 16, 16)] = jnp.zeros((16,), _f32)
        return carry

    lax.fori_loop(0, SR, zrow, 0)
    for t in range(6):
        r0 = pl.multiple_of(sid * RPT + t * SR, 8)
        pltpu.sync_copy(stage, acc_sh.at[pl.ds(r0, SR)])

    @pl.when(sid == 0)
    def _():
        pltpu.sync_copy(stage.at[pl.ds(0, 16)],
                        acc_sh.at[pl.ds(NS * RPT, N - NS * RPT)])

    def load_and_fire(c, b):
        off = pl.multiple_of(ebase + c * CH, 8)
        pltpu.sync_copy(ridx_hbm.at[pl.ds(off, CH)], rbuf[b])
        pltpu.sync_copy(cidx_hbm.at[pl.ds(off, CH)], cbuf[b])
        pltpu.sync_copy(coef_hbm.at[pl.ds(off, CH)], fbuf[b])
        pltpu.async_copy(h_hbm.at[rbuf[b]], rows[b], sems[b])

    def process(b):
        pltpu.make_async_copy(h_hbm.at[rbuf[b]], rows[b], sems[b]).wait()

        def grp(u, carry):
            u16 = pl.multiple_of(u * 16, 16)
            nv = fbuf[b][pl.ds(u16, 16)]
            for e16 in range(16):
                w = nv.at[jnp.full((16,), e16, jnp.int32)].get(
                    mode="promise_in_bounds")
                e = u16 + e16
                for k in range(DG // 16):
                    rows[b][e, pl.ds(k * 16, 16)] = (
                        rows[b][e, pl.ds(k * 16, 16)] * w)
            c16 = cbuf[b][pl.ds(u16, 16)]
            pltpu.sync_copy(rows[b].at[pl.ds(u16, 16)],
                            acc_sh.at[c16], add=True)
            return carry

        lax.fori_loop(0, CH // 16, grp, 0)

    plsc.subcore_barrier()
    for b in range(2):
        load_and_fire(b, b)

    def outer(j, carry):
        for b in range(2):
            c = j * 2 + b
            process(b)

            @pl.when(c + 2 < NCHK)
            def _():
                load_and_fire(c + 2, b)
        return carry

    lax.fori_loop(0, NCHK // 2, outer, 0)
    process(0)

    plsc.subcore_barrier()
    for t in range(6):
        r0 = pl.multiple_of(sid * RPT + t * SR, 8)
        pltpu.sync_copy(acc_sh.at[pl.ds(r0, SR)], stage)
        pltpu.sync_copy(stage, accp_hbm.at[pl.ds(cid * N + r0, SR)])

    @pl.when(sid == 0)
    def _():
        tail = pl.multiple_of(NS * RPT, 8)
        pltpu.sync_copy(acc_sh.at[pl.ds(tail, N - NS * RPT)],
                        stage.at[pl.ds(0, 16)])
        pltpu.sync_copy(stage.at[pl.ds(0, 16)],
                        accp_hbm.at[pl.ds(cid * N + tail, N - NS * RPT)])


def _tc(body, out_shape, *args):
    return pl.pallas_call(body, out_shape=out_shape)(*args)


def _mm_body(x_ref, w_ref, o_ref):
    o_ref[...] = jnp.dot(x_ref[...], w_ref[...], preferred_element_type=_f32)


def _dinv_body(degp_ref, o_ref):
    deg = jnp.sum(degp_ref[...], axis=0) + 1.0
    o_ref[...] = lax.rsqrt(deg)


def _comb_mm_body(accp_ref, h_ref, dinv_ref, b_ref, w_ref, o_ref):
    dinv2 = (dinv_ref[...] * dinv_ref[...])[:, None]
    out = accp_ref[0] + accp_ref[1] + dinv2 * h_ref[...] + b_ref[...][None, :]
    o_ref[...] = jnp.dot(out, w_ref[...], preferred_element_type=_f32)


def _gat_mm_body(accp_ref, h_ref, dinv_ref, b_ref, w_ref, asrc_ref, adst_ref,
                 hg_ref, as_ref, ad_ref):
    dinv2 = (dinv_ref[...] * dinv_ref[...])[:, None]
    out = accp_ref[0] + accp_ref[1] + dinv2 * h_ref[...] + b_ref[...][None, :]
    hg = jnp.dot(out, w_ref[...], preferred_element_type=_f32)
    hg_ref[...] = hg
    as_ref[...] = jnp.dot(hg, asrc_ref[...], preferred_element_type=_f32)
    ad_ref[...] = jnp.dot(hg, adst_ref[...], preferred_element_type=_f32)


def _shift_body(as_ref, ad_ref, p_ref, eself_ref):
    amax = jnp.max(as_ref[...])
    pa = amax + ad_ref[...]
    p = jnp.where(pa > 0, pa, 0.2 * pa)
    p_ref[...] = p
    al = as_ref[...] + ad_ref[...]
    al = jnp.where(al > 0, al, 0.2 * al)
    eself_ref[...] = jnp.exp(al - p)


def _sinv_body(sp_ref, eself_ref, sinv_ref, cself_ref):
    s = jnp.sum(sp_ref[...], axis=0) + eself_ref[...]
    sinv = 1.0 / (s + 1e-16)
    sinv_ref[...] = sinv
    cself_ref[...] = eself_ref[...] * sinv


def _final_body(accg_ref, hg_ref, cself_ref, bg_ref, o_ref):
    o_ref[...] = (accg_ref[0] + accg_ref[1]
                  + cself_ref[...][:, None] * hg_ref[...]
                  + bg_ref[...][None, :])


def kernel(x, edge_index, edge_attr, W1, b1, W2, b2, W3, b3, Wg,
           att_src, att_dst, bg):
    ridx = edge_index[0]
    cidx = edge_index[1]
    zn = jnp.zeros((N,), _f32)
    pad = DG - DH
    W1p = jnp.pad(W1, ((0, 0), (0, pad)))
    W2p = jnp.pad(W2, ((0, pad), (0, pad)))
    W3p = jnp.pad(W3, ((0, pad), (0, pad)))
    Wgp = jnp.pad(Wg, ((0, pad), (0, 0)))
    b1p = jnp.pad(b1, (0, pad))
    b2p = jnp.pad(b2, (0, pad))
    b3p = jnp.pad(b3, (0, pad))

    degp = _sc_deg(cidx, edge_attr, zn).reshape(NW, N)
    dinv = _tc(_dinv_body, jax.ShapeDtypeStruct((N,), _f32), degp)

    norm = _sc_norm(ridx, cidx, edge_attr, dinv)

    h1 = _tc(_mm_body, jax.ShapeDtypeStruct((N, DG), _f32), x, W1p)
    acc1 = _sc_row(h1, ridx, cidx, norm).reshape(NC, N, DG)
    h2 = _tc(_comb_mm_body, jax.ShapeDtypeStruct((N, DG), _f32),
             acc1, h1, dinv, b1p, W2p)
    acc2 = _sc_row(h2, ridx, cidx, norm).reshape(NC, N, DG)
    h3 = _tc(_comb_mm_body, jax.ShapeDtypeStruct((N, DG), _f32),
             acc2, h2, dinv, b2p, W3p)
    acc3 = _sc_row(h3, ridx, cidx, norm).reshape(NC, N, DG)

    hg, a_s, a_d = _tc(_gat_mm_body,
                       [jax.ShapeDtypeStruct((N, DG), _f32),
                        jax.ShapeDtypeStruct((N,), _f32),
                        jax.ShapeDtypeStruct((N,), _f32)],
                       acc3, h3, dinv, b3p, Wgp, att_src, att_dst)
    p, e_self = _tc(_shift_body,
                    [jax.ShapeDtypeStruct((N,), _f32),
                     jax.ShapeDtypeStruct((N,), _f32)],
                    a_s, a_d)
    e_s, s_p = _sc_att(ridx, cidx, a_s, a_d, p, zn)
    sinv, cself = _tc(_sinv_body,
                      [jax.ShapeDtypeStruct((N,), _f32),
                       jax.ShapeDtypeStruct((N,), _f32)],
                      s_p.reshape(NW, N), e_self)
    coefg = _sc_coef(cidx, e_s, sinv)
    accg = _sc_row(hg, ridx, cidx, coefg).reshape(NC, N, DG)
    out = _tc(_final_body, jax.ShapeDtypeStruct((N, DG), _f32),
              accg, hg, cself, bg)
    return out
